# baseline (speedup 1.0000x reference)
import functools

import jax
import jax.numpy as jnp
from jax import lax
from jax.experimental import pallas as pl
from jax.experimental.pallas import tpu as pltpu
from jax.experimental.pallas import tpu_sc as plsc

_B = 4096
_CTX = 20
_D = 128
_LANES = 16
_CHUNK = 16


def _sc_gather_sum(emb_table, idx_flat):
    mesh = plsc.VectorSubcoreMesh(core_axis_name="c", subcore_axis_name="s")
    nw = mesh.num_cores * mesh.num_subcores
    b_per_w = _B // nw
    n_chunks = b_per_w // _CHUNK
    rows_per_chunk = _CHUNK * _CTX

    @functools.partial(
        pl.kernel,
        out_type=jax.ShapeDtypeStruct((_B, _D), jnp.float32),
        mesh=mesh,
        scratch_types=[
            pltpu.VMEM((b_per_w * _CTX,), jnp.int32),
            pltpu.VMEM((rows_per_chunk, _D), jnp.float32),
            pltpu.VMEM((rows_per_chunk, _D), jnp.float32),
            pltpu.VMEM((b_per_w, _D), jnp.float32),
            pltpu.SemaphoreType.DMA,
            pltpu.SemaphoreType.DMA,
        ],
    )
    def k(table_hbm, idx_hbm, out_hbm, idx_all, rows0, rows1, acc_v, sem0, sem1):
        wid = lax.axis_index("s") * mesh.num_cores + lax.axis_index("c")
        ibase = wid * (b_per_w * _CTX)
        rows = (rows0, rows1)
        sems = (sem0, sem1)
        pltpu.sync_copy(idx_hbm.at[pl.ds(ibase, b_per_w * _CTX)], idx_all)

        def start(c, slot):
            return pltpu.async_copy(
                table_hbm.at[idx_all.at[pl.ds(c * rows_per_chunk,
                                              rows_per_chunk)]],
                rows[slot], sems[slot])

        pending = {0: start(0, 0)}
        for c in range(n_chunks):
            s = c & 1
            if c + 1 < n_chunks:
                pending[c + 1] = start(c + 1, (c + 1) & 1)
            pending.pop(c).wait()
            rbuf = rows[s]

            def body(r, carry, c=c, rbuf=rbuf):
                accs = [rbuf[r * _CTX, pl.ds(d * _LANES, _LANES)]
                        for d in range(_D // _LANES)]
                for j in range(1, _CTX):
                    for d in range(_D // _LANES):
                        accs[d] = accs[d] + rbuf[r * _CTX + j,
                                                 pl.ds(d * _LANES, _LANES)]
                for d in range(_D // _LANES):
                    acc_v[c * _CHUNK + r, pl.ds(d * _LANES, _LANES)] = accs[d]
                return carry

            lax.fori_loop(0, _CHUNK, body, 0)

        pltpu.sync_copy(acc_v, out_hbm.at[pl.ds(wid * b_per_w, b_per_w)])

    return k(emb_table, idx_flat)


_BN = 1536
_N = 100001
_NT = (_N + _BN - 1) // _BN


def _tc_body(xt_ref, praw_ref, emb0_ref, w_ref, b_ref, out_ref, pct_ref):
    @pl.when(pl.program_id(0) == 0)
    def _():
        z = jnp.sum((xt_ref[...] == 0).astype(jnp.float32), axis=0,
                    keepdims=True).T
        pc = (praw_ref[...] - z * emb0_ref[0:1, :]) * (1.0 / _CTX)
        pct_ref[...] = pc.astype(jnp.bfloat16).T

    out_ref[...] = lax.dot_general(
        w_ref[...].astype(jnp.bfloat16), pct_ref[...],
        (((1,), (0,)), ((), ())),
        preferred_element_type=jnp.float32,
    ) + b_ref[...]


def _tc_project(x, pooled_raw, emb_table, w_out, b_out):
    n = w_out.shape[0]
    out_t = pl.pallas_call(
        _tc_body,
        grid=(_NT,),
        in_specs=[
            pl.BlockSpec((_CTX, _B), lambda i: (0, 0)),
            pl.BlockSpec((_B, _D), lambda i: (0, 0)),
            pl.BlockSpec((8, _D), lambda i: (0, 0)),
            pl.BlockSpec((_BN, _D), lambda i: (i, 0)),
            pl.BlockSpec((_BN, 1), lambda i: (i, 0)),
        ],
        out_specs=pl.BlockSpec((_BN, _B), lambda i: (i, 0)),
        out_shape=jax.ShapeDtypeStruct((n, _B), jnp.float32),
        scratch_shapes=[pltpu.VMEM((_D, _B), jnp.bfloat16)],
    )(x.T, pooled_raw, emb_table, w_out, b_out.reshape(n, 1))
    return out_t.T


def kernel(x, emb_table, W_out, b_out):
    x = x.astype(jnp.int32)
    idx_flat = x.reshape(-1)
    pooled_raw = _sc_gather_sum(emb_table, idx_flat)
    return _tc_project(x, pooled_raw, emb_table, W_out, b_out)


# speedup vs baseline: 1.0019x; 1.0019x over previous
import functools

import jax
import jax.numpy as jnp
from jax import lax
from jax.experimental import pallas as pl
from jax.experimental.pallas import tpu as pltpu
from jax.experimental.pallas import tpu_sc as plsc

_B = 4096
_CTX = 20
_D = 128
_LANES = 16
_CHUNK = 8
_SC_NBUF = 4


def _sc_gather_sum(emb_table, idx_flat):
    mesh = plsc.VectorSubcoreMesh(core_axis_name="c", subcore_axis_name="s")
    nw = mesh.num_cores * mesh.num_subcores
    b_per_w = _B // nw
    n_chunks = b_per_w // _CHUNK
    rows_per_chunk = _CHUNK * _CTX

    @functools.partial(
        pl.kernel,
        out_type=jax.ShapeDtypeStruct((_B, _D), jnp.float32),
        mesh=mesh,
        scratch_types=(
            [pltpu.VMEM((b_per_w * _CTX,), jnp.int32)]
            + [pltpu.VMEM((rows_per_chunk, _D), jnp.float32)
               for _ in range(_SC_NBUF)]
            + [pltpu.VMEM((b_per_w, _D), jnp.float32)]
            + [pltpu.SemaphoreType.DMA for _ in range(_SC_NBUF)]
        ),
    )
    def k(table_hbm, idx_hbm, out_hbm, idx_all, *rest):
        rows = rest[:_SC_NBUF]
        acc_v = rest[_SC_NBUF]
        sems = rest[_SC_NBUF + 1:]
        wid = lax.axis_index("s") * mesh.num_cores + lax.axis_index("c")
        ibase = wid * (b_per_w * _CTX)
        pltpu.sync_copy(idx_hbm.at[pl.ds(ibase, b_per_w * _CTX)], idx_all)

        def start(c):
            slot = c % _SC_NBUF
            return pltpu.async_copy(
                table_hbm.at[idx_all.at[pl.ds(c * rows_per_chunk,
                                              rows_per_chunk)]],
                rows[slot], sems[slot])

        pending = {c: start(c) for c in range(_SC_NBUF - 1)}
        for c in range(n_chunks):
            s = c % _SC_NBUF
            if c + _SC_NBUF - 1 < n_chunks:
                pending[c + _SC_NBUF - 1] = start(c + _SC_NBUF - 1)
            pending.pop(c).wait()
            rbuf = rows[s]

            def body(r, carry, c=c, rbuf=rbuf):
                accs = [rbuf[r * _CTX, pl.ds(d * _LANES, _LANES)]
                        for d in range(_D // _LANES)]
                for j in range(1, _CTX):
                    for d in range(_D // _LANES):
                        accs[d] = accs[d] + rbuf[r * _CTX + j,
                                                 pl.ds(d * _LANES, _LANES)]
                for d in range(_D // _LANES):
                    acc_v[c * _CHUNK + r, pl.ds(d * _LANES, _LANES)] = accs[d]
                return carry

            lax.fori_loop(0, _CHUNK, body, 0)

        pltpu.sync_copy(acc_v, out_hbm.at[pl.ds(wid * b_per_w, b_per_w)])

    return k(emb_table, idx_flat)


_BN = 1536
_N = 100001
_NT = (_N + _BN - 1) // _BN


def _tc_body(xt_ref, praw_ref, emb0_ref, w_ref, b_ref, out_ref, pct_ref):
    @pl.when(pl.program_id(0) == 0)
    def _():
        z = jnp.sum((xt_ref[...] == 0).astype(jnp.float32), axis=0,
                    keepdims=True).T
        pc = (praw_ref[...] - z * emb0_ref[0:1, :]) * (1.0 / _CTX)
        pct_ref[...] = pc.astype(jnp.bfloat16).T

    out_ref[...] = lax.dot_general(
        w_ref[...].astype(jnp.bfloat16), pct_ref[...],
        (((1,), (0,)), ((), ())),
        preferred_element_type=jnp.float32,
    ) + b_ref[...]


def _tc_project(x, pooled_raw, emb_table, w_out, b_out):
    n = w_out.shape[0]
    out_t = pl.pallas_call(
        _tc_body,
        grid=(_NT,),
        in_specs=[
            pl.BlockSpec((_CTX, _B), lambda i: (0, 0)),
            pl.BlockSpec((_B, _D), lambda i: (0, 0)),
            pl.BlockSpec((8, _D), lambda i: (0, 0)),
            pl.BlockSpec((_BN, _D), lambda i: (i, 0)),
            pl.BlockSpec((_BN, 1), lambda i: (i, 0)),
        ],
        out_specs=pl.BlockSpec((_BN, _B), lambda i: (i, 0)),
        out_shape=jax.ShapeDtypeStruct((n, _B), jnp.float32),
        scratch_shapes=[pltpu.VMEM((_D, _B), jnp.bfloat16)],
    )(x.T, pooled_raw, emb_table, w_out, b_out.reshape(n, 1))
    return out_t.T


def kernel(x, emb_table, W_out, b_out):
    x = x.astype(jnp.int32)
    idx_flat = x.reshape(-1)
    pooled_raw = _sc_gather_sum(emb_table, idx_flat)
    return _tc_project(x, pooled_raw, emb_table, W_out, b_out)
